# baseline (speedup 1.0000x reference)
import functools

import jax
import jax.numpy as jnp
from jax import lax
from jax.experimental import pallas as pl
from jax.experimental.pallas import tpu as pltpu
from jax.experimental.pallas import tpu_sc as plsc

NC = 2
NS = 16
NW = NC * NS
CHUNK = 128
NSPLIT = 1

_HI = jax.lax.Precision.DEFAULT


def _ln(t, g, b, eps=1e-5):
    m = jnp.mean(t, axis=-1, keepdims=True)
    c = t - m
    v = jnp.mean(c * c, axis=-1, keepdims=True)
    return c * lax.rsqrt(v + eps) * g + b


def _pk_body(x_ref, w_ref, o_ref):
    o_ref[...] = jnp.dot(x_ref[...], w_ref[...], precision=_HI)


def _stage_a(x_h, w1a, blk):
    n, d = x_h.shape
    return pl.pallas_call(
        _pk_body,
        grid=(n // blk,),
        in_specs=[
            pl.BlockSpec((blk, d), lambda i: (i, 0)),
            pl.BlockSpec((d, d), lambda i: (0, 0)),
        ],
        out_specs=pl.BlockSpec((blk, d), lambda i: (i, 0)),
        out_shape=jax.ShapeDtypeStruct((n, d), jnp.float32),
    )(x_h, w1a)


def _gather_body(nfull, rem, p_hbm, row3_hbm, g_hbm, idx_v, buf0, buf1,
                 sem0, sem1, osem0, osem1):
    d = p_hbm.shape[1]
    w = lax.axis_index("c") * NS + lax.axis_index("s")
    n = nfull + jnp.where(w < rem, 1, 0)
    pltpu.sync_copy(row3_hbm.at[w], idx_v)

    bufs = (buf0, buf1)
    sems = (sem0, sem1)
    osems = (osem0, osem1)

    def out_slice(i):
        cid = w + NW * i
        return g_hbm.at[pl.ds(cid * CHUNK, CHUNK)]

    def start(i, b):
        @pl.when(i >= 2)
        def _():
            pltpu.make_async_copy(bufs[b], out_slice(i - 2), osems[b]).wait()

        pltpu.async_copy(p_hbm.at[idx_v.at[i]], bufs[b], sems[b])

    def finish(i, b):
        pltpu.make_async_copy(p_hbm.at[idx_v.at[i]], bufs[b], sems[b]).wait()
        pltpu.async_copy(bufs[b], out_slice(i), osems[b])

    def drain(b):
        ib = n - 1 - ((n - 1 - b) % 2)

        @pl.when(ib >= 0)
        def _():
            pltpu.make_async_copy(bufs[b], out_slice(ib), osems[b]).wait()

    pltpu.async_copy(p_hbm.at[idx_v.at[0]], bufs[0], sems[0])

    def pair(t, carry):
        i0 = 2 * t
        i1 = i0 + 1

        @pl.when(i1 < n)
        def _():
            start(i1, 1)

        @pl.when(i0 < n)
        def _():
            finish(i0, 0)

        @pl.when(i1 < n)
        def _():
            @pl.when(i1 + 1 < n)
            def _():
                start(i1 + 1, 0)

            finish(i1, 1)

        return carry

    nt = (nfull + 2) // 2
    lax.fori_loop(0, nt, pair, 0, unroll=False)
    drain(0)
    drain(1)


def _stage_b(p, row3, e, nfull, rem):
    d = p.shape[1]
    cw = row3.shape[1]
    gk = pl.kernel(
        functools.partial(_gather_body, nfull, rem),
        out_type=[jax.ShapeDtypeStruct((e, d), jnp.float32)],
        mesh=plsc.VectorSubcoreMesh(
            core_axis_name="c", subcore_axis_name="s", num_cores=NC, num_subcores=NS
        ),
        scratch_types=[
            pltpu.VMEM((cw, CHUNK), jnp.int32),
            pltpu.VMEM((CHUNK, d), jnp.float32),
            pltpu.VMEM((CHUNK, d), jnp.float32),
            pltpu.SemaphoreType.DMA,
            pltpu.SemaphoreType.DMA,
            pltpu.SemaphoreType.DMA,
            pltpu.SemaphoreType.DMA,
        ],
    )
    return gk(p, row3)[0]


def _edge_body(g_ref, ea_ref, w_ref, b_ref, g1_ref, be1_ref, o_ref):
    q = jnp.dot(ea_ref[...].astype(jnp.bfloat16), w_ref[...],
                preferred_element_type=jnp.float32)
    t = g_ref[...] + q + b_ref[...]
    t = jnp.maximum(t, 0.0)
    m = jnp.mean(t, axis=-1, keepdims=True)
    ms = jnp.mean(t * t, axis=-1, keepdims=True)
    r = lax.rsqrt(jnp.maximum(ms - m * m, 0.0) + 1e-5)
    o_ref[...] = (t - m) * (r * g1_ref[...]) + be1_ref[...]


def _stage_c(g, ea, w1b, b1, g1, be1, blk, off=0):
    e, d = g.shape
    ob = off // blk
    return pl.pallas_call(
        _edge_body,
        grid=(e // blk,),
        in_specs=[
            pl.BlockSpec((blk, d), lambda i: (i, 0)),
            pl.BlockSpec((blk, d), lambda i: (ob + i, 0)),
            pl.BlockSpec((d, d), lambda i: (0, 0)),
            pl.BlockSpec((1, d), lambda i: (0, 0)),
            pl.BlockSpec((1, d), lambda i: (0, 0)),
            pl.BlockSpec((1, d), lambda i: (0, 0)),
        ],
        out_specs=pl.BlockSpec((blk, d), lambda i: (i, 0)),
        out_shape=jax.ShapeDtypeStruct((e, d), jnp.float32),
    )(g, ea, w1b, b1, g1, be1)


def _mk_idx(idx_flat, chunks):
    nfull, rem = divmod(chunks, NW)
    cw = nfull + (1 if rem else 0)
    pad = NW * cw - chunks
    ipad = jnp.concatenate([idx_flat, jnp.zeros((pad * CHUNK,), jnp.int32)])
    return ipad.reshape(cw, NW, CHUNK).transpose(1, 0, 2), nfull, rem


def _scatter_body(nfull, rem, n_pad, h_hbm, col4_hbm, zer_hbm, out_hbm,
                  cnt_hbm, ibuf0, ibuf1, buf0, buf1, cnt_v, acc, sem0, sem1,
                  ssem0, ssem1):
    d = h_hbm.shape[1]
    c = lax.axis_index("c")
    s = lax.axis_index("s")
    w = c * NS + s
    n = nfull + jnp.where(w < rem, 1, 0)
    rows_per = n_pad // NS

    pltpu.sync_copy(zer_hbm.at[pl.ds(s * rows_per, rows_per)],
                    acc.at[pl.ds(s * rows_per, rows_per)])

    z16 = jnp.zeros((16,), jnp.float32)

    def zb(i, carry):
        cnt_v[pl.ds(i * 16, 16)] = z16
        return carry

    lax.fori_loop(0, n_pad // 16, zb, 0, unroll=False)
    plsc.subcore_barrier()

    bufs = (buf0, buf1)
    ibufs = (ibuf0, ibuf1)
    sems = (sem0, sem1)
    ssems = (ssem0, ssem1)
    one16 = jnp.ones((16,), jnp.float32)

    def start(i, b):
        @pl.when(i >= 2)
        def _():
            pltpu.make_async_copy(bufs[b], acc.at[ibufs[b].at[0]],
                                  ssems[b]).wait()

        cid = w + NW * i
        pltpu.async_copy(col4_hbm.at[cid], ibufs[b], sems[b])
        pltpu.async_copy(h_hbm.at[pl.ds(cid * CHUNK, CHUNK)], bufs[b], sems[b])

    def finish(i, b):
        cid = w + NW * i
        pltpu.make_async_copy(col4_hbm.at[cid], ibufs[b], sems[b]).wait()
        pltpu.make_async_copy(
            h_hbm.at[pl.ds(cid * CHUNK, CHUNK)], bufs[b], sems[b]
        ).wait()
        pltpu.async_copy(bufs[b], acc.at[ibufs[b].at[0]], ssems[b], add=True)
        for k in range(CHUNK // 16):
            iv = ibufs[b][0, pl.ds(k * 16, 16)]
            plsc.addupdate_scatter(cnt_v, [iv], one16)

    def drain(b):
        ib = n - 1 - ((n - 1 - b) % 2)

        @pl.when(ib >= 0)
        def _():
            pltpu.make_async_copy(bufs[b], acc.at[ibufs[b].at[0]],
                                  ssems[b]).wait()

    cid0 = w
    pltpu.async_copy(col4_hbm.at[cid0], ibufs[0], sems[0])
    pltpu.async_copy(h_hbm.at[pl.ds(cid0 * CHUNK, CHUNK)], bufs[0], sems[0])

    def pair(t, carry):
        i0 = 2 * t
        i1 = i0 + 1

        @pl.when(i1 < n)
        def _():
            start(i1, 1)

        @pl.when(i0 < n)
        def _():
            finish(i0, 0)

        @pl.when(i1 < n)
        def _():
            @pl.when(i1 + 1 < n)
            def _():
                start(i1 + 1, 0)

            finish(i1, 1)

        return carry

    nt = (nfull + 2) // 2
    lax.fori_loop(0, nt, pair, 0, unroll=False)
    drain(0)
    drain(1)

    plsc.subcore_barrier()
    pltpu.sync_copy(acc.at[pl.ds(s * rows_per, rows_per)],
                    out_hbm.at[c, pl.ds(s * rows_per, rows_per)])
    pltpu.sync_copy(cnt_v, cnt_hbm.at[c, s])


def _stage_d(h, col4, n_nodes, nfull, rem):
    d = h.shape[1]
    n_pad = ((n_nodes + 8 * NS - 1) // (8 * NS)) * 8 * NS
    sk = pl.kernel(
        functools.partial(_scatter_body, nfull, rem, n_pad),
        out_type=[
            jax.ShapeDtypeStruct((NC, n_pad, d), jnp.float32),
            jax.ShapeDtypeStruct((NC, NS, n_pad), jnp.float32),
        ],
        mesh=plsc.VectorSubcoreMesh(
            core_axis_name="c", subcore_axis_name="s", num_cores=NC, num_subcores=NS
        ),
        scratch_types=[
            pltpu.VMEM((1, CHUNK), jnp.int32),
            pltpu.VMEM((1, CHUNK), jnp.int32),
            pltpu.VMEM((CHUNK, d), jnp.float32),
            pltpu.VMEM((CHUNK, d), jnp.float32),
            pltpu.VMEM((n_pad,), jnp.float32),
            pltpu.VMEM_SHARED((n_pad, d), jnp.float32),
            pltpu.SemaphoreType.DMA,
            pltpu.SemaphoreType.DMA,
            pltpu.SemaphoreType.DMA,
            pltpu.SemaphoreType.DMA,
        ],
        compiler_params=pltpu.CompilerParams(needs_layout_passes=False),
    )
    zeros = jnp.zeros((n_pad, d), jnp.float32)
    return sk(h, col4, zeros)


def _node_body(nparts, *refs):
    parts_refs = refs[:nparts]
    (cnt_ref, x_ref, bat_ref, u_ref, w2a_ref, w2b_ref, w2c_ref, b2_ref,
     g2_ref, be2_ref, w3_ref, b3_ref, g3_ref, be3_ref, o_ref) = refs[nparts:]
    blk, d = x_ref.shape
    nb = u_ref.shape[1]
    seg = parts_refs[0][0] + parts_refs[0][1]
    for pr in parts_refs[1:]:
        seg = seg + pr[0] + pr[1]
    cnt = jnp.sum(cnt_ref[...], axis=1, keepdims=True)
    agg = seg / jnp.maximum(cnt, 1.0)
    x = x_ref[...]
    oh = (lax.broadcasted_iota(jnp.int32, (blk, nb), 1) == bat_ref[...]).astype(
        jnp.float32
    )
    ub = jnp.sum(oh * u_ref[...], axis=1, keepdims=True)
    t = (
        jnp.dot(x, w2a_ref[...], precision=_HI)
        + jnp.dot(agg, w2b_ref[...], precision=_HI)
        + ub * w2c_ref[...]
        + b2_ref[...]
    )
    t = jnp.maximum(t, 0.0)
    y = _ln(t, g2_ref[...], be2_ref[...])
    y = jnp.dot(y, w3_ref[...], precision=_HI) + b3_ref[...]
    o_ref[...] = _ln(y, g3_ref[...], be3_ref[...]) + x


def _stage_e(parts_list, cnt_t, x_h, bat2, u_row, w2a, w2b, w2c, b2, g2, be2,
             w3, b3, g3, be3, blk):
    n, d = x_h.shape
    nw = cnt_t.shape[1]
    nb = u_row.shape[1]
    nparts = len(parts_list)
    full = lambda i: (0, 0)
    return pl.pallas_call(
        functools.partial(_node_body, nparts),
        grid=(n // blk,),
        in_specs=[
            pl.BlockSpec((NC, blk, d), lambda i: (0, i, 0))
            for _ in range(nparts)
        ] + [
            pl.BlockSpec((blk, nw), lambda i: (i, 0)),
            pl.BlockSpec((blk, d), lambda i: (i, 0)),
            pl.BlockSpec((blk, 1), lambda i: (i, 0)),
            pl.BlockSpec((1, nb), full),
            pl.BlockSpec((d, d), full),
            pl.BlockSpec((d, d), full),
            pl.BlockSpec((1, d), full),
            pl.BlockSpec((1, d), full),
            pl.BlockSpec((1, d), full),
            pl.BlockSpec((1, d), full),
            pl.BlockSpec((d, d), full),
            pl.BlockSpec((1, d), full),
            pl.BlockSpec((1, d), full),
            pl.BlockSpec((1, d), full),
        ],
        out_specs=pl.BlockSpec((blk, d), lambda i: (i, 0)),
        out_shape=jax.ShapeDtypeStruct((n, d), jnp.float32),
    )(*parts_list, cnt_t, x_h, bat2, u_row, w2a, w2b, w2c, b2, g2, be2, w3,
      b3, g3, be3)


def kernel(x_h, edge_index, edge_attr_h, u, batch,
           W1, b1, g1, be1, W2, b2, g2, be2, W3, b3, g3, be3):
    n, d = x_h.shape
    e = edge_index.shape[1]
    nb = u.shape[0]

    row = edge_index[0]
    col = edge_index[1]

    w1a, w1b = W1[:d], W1[d:]
    w2a, w2b, w2c = W2[:d], W2[d:2 * d], W2[2 * d:2 * d + 1]
    r = lambda v: v.reshape(1, d)

    p = _stage_a(x_h, w1a, 2000)

    nsplit = NSPLIT
    eh = e // nsplit
    parts_list, cnts_list = [], []
    for si in range(nsplit):
        a = si * eh
        row3, nfull, rem = _mk_idx(lax.dynamic_slice(row, (a,), (eh,)), eh // CHUNK)
        col4 = lax.dynamic_slice(col, (a,), (eh,)).reshape(eh // CHUNK, 1, CHUNK)
        g = _stage_b(p, row3, eh, nfull, rem)
        h = _stage_c(g, edge_attr_h, w1b.astype(jnp.bfloat16), r(b1), r(g1),
                     r(be1), 2000, off=a)
        parts, cnts = _stage_d(h, col4, n, nfull, rem)
        parts_list.append(parts)
        cnts_list.append(cnts)

    cnt_t = jnp.concatenate([c.reshape(NW, -1) for c in cnts_list], axis=0).T
    out = _stage_e(parts_list, cnt_t, x_h, batch.reshape(n, 1),
                   u.reshape(1, nb), w2a, w2b, w2c, r(b2), r(g2), r(be2), W3,
                   r(b3), r(g3), r(be3), 2000)
    return out


# speedup vs baseline: 1.0800x; 1.0800x over previous
import functools

import jax
import jax.numpy as jnp
from jax import lax
from jax.experimental import pallas as pl
from jax.experimental.pallas import tpu as pltpu
from jax.experimental.pallas import tpu_sc as plsc

NC = 2
NS = 16
NW = NC * NS
CHUNK = 128
NSPLIT = 2

_HI = jax.lax.Precision.DEFAULT


def _ln(t, g, b, eps=1e-5):
    m = jnp.mean(t, axis=-1, keepdims=True)
    c = t - m
    v = jnp.mean(c * c, axis=-1, keepdims=True)
    return c * lax.rsqrt(v + eps) * g + b


def _pk_body(x_ref, w_ref, o_ref):
    o_ref[...] = jnp.dot(x_ref[...], w_ref[...], precision=_HI)


def _stage_a(x_h, w1a, blk):
    n, d = x_h.shape
    return pl.pallas_call(
        _pk_body,
        grid=(n // blk,),
        in_specs=[
            pl.BlockSpec((blk, d), lambda i: (i, 0)),
            pl.BlockSpec((d, d), lambda i: (0, 0)),
        ],
        out_specs=pl.BlockSpec((blk, d), lambda i: (i, 0)),
        out_shape=jax.ShapeDtypeStruct((n, d), jnp.float32),
    )(x_h, w1a)


def _gather_body(nfull, rem, p_hbm, row3_hbm, g_hbm, idx_v, buf0, buf1,
                 sem0, sem1, osem0, osem1):
    d = p_hbm.shape[1]
    w = lax.axis_index("c") * NS + lax.axis_index("s")
    n = nfull + jnp.where(w < rem, 1, 0)
    pltpu.sync_copy(row3_hbm.at[w], idx_v)

    bufs = (buf0, buf1)
    sems = (sem0, sem1)
    osems = (osem0, osem1)

    def out_slice(i):
        cid = w + NW * i
        return g_hbm.at[pl.ds(cid * CHUNK, CHUNK)]

    def start(i, b):
        @pl.when(i >= 2)
        def _():
            pltpu.make_async_copy(bufs[b], out_slice(i - 2), osems[b]).wait()

        pltpu.async_copy(p_hbm.at[idx_v.at[i]], bufs[b], sems[b])

    def finish(i, b):
        pltpu.make_async_copy(p_hbm.at[idx_v.at[i]], bufs[b], sems[b]).wait()
        pltpu.async_copy(bufs[b], out_slice(i), osems[b])

    def drain(b):
        ib = n - 1 - ((n - 1 - b) % 2)

        @pl.when(ib >= 0)
        def _():
            pltpu.make_async_copy(bufs[b], out_slice(ib), osems[b]).wait()

    pltpu.async_copy(p_hbm.at[idx_v.at[0]], bufs[0], sems[0])

    def pair(t, carry):
        i0 = 2 * t
        i1 = i0 + 1

        @pl.when(i1 < n)
        def _():
            start(i1, 1)

        @pl.when(i0 < n)
        def _():
            finish(i0, 0)

        @pl.when(i1 < n)
        def _():
            @pl.when(i1 + 1 < n)
            def _():
                start(i1 + 1, 0)

            finish(i1, 1)

        return carry

    nt = (nfull + 2) // 2
    lax.fori_loop(0, nt, pair, 0, unroll=False)
    drain(0)
    drain(1)


def _stage_b(p, row3, e, nfull, rem):
    d = p.shape[1]
    cw = row3.shape[1]
    gk = pl.kernel(
        functools.partial(_gather_body, nfull, rem),
        out_type=[jax.ShapeDtypeStruct((e, d), jnp.float32)],
        mesh=plsc.VectorSubcoreMesh(
            core_axis_name="c", subcore_axis_name="s", num_cores=NC, num_subcores=NS
        ),
        scratch_types=[
            pltpu.VMEM((cw, CHUNK), jnp.int32),
            pltpu.VMEM((CHUNK, d), jnp.float32),
            pltpu.VMEM((CHUNK, d), jnp.float32),
            pltpu.SemaphoreType.DMA,
            pltpu.SemaphoreType.DMA,
            pltpu.SemaphoreType.DMA,
            pltpu.SemaphoreType.DMA,
        ],
    )
    return gk(p, row3)[0]


def _edge_body(g_ref, ea_ref, w_ref, b_ref, g1_ref, be1_ref, o_ref):
    q = jnp.dot(ea_ref[...].astype(jnp.bfloat16), w_ref[...],
                preferred_element_type=jnp.float32)
    t = g_ref[...] + q + b_ref[...]
    t = jnp.maximum(t, 0.0)
    m = jnp.mean(t, axis=-1, keepdims=True)
    ms = jnp.mean(t * t, axis=-1, keepdims=True)
    r = lax.rsqrt(jnp.maximum(ms - m * m, 0.0) + 1e-5)
    o_ref[...] = (t - m) * (r * g1_ref[...]) + be1_ref[...]


def _stage_c(g, ea, w1b, b1, g1, be1, blk, off=0):
    e, d = g.shape
    ob = off // blk
    return pl.pallas_call(
        _edge_body,
        grid=(e // blk,),
        in_specs=[
            pl.BlockSpec((blk, d), lambda i: (i, 0)),
            pl.BlockSpec((blk, d), lambda i: (ob + i, 0)),
            pl.BlockSpec((d, d), lambda i: (0, 0)),
            pl.BlockSpec((1, d), lambda i: (0, 0)),
            pl.BlockSpec((1, d), lambda i: (0, 0)),
            pl.BlockSpec((1, d), lambda i: (0, 0)),
        ],
        out_specs=pl.BlockSpec((blk, d), lambda i: (i, 0)),
        out_shape=jax.ShapeDtypeStruct((e, d), jnp.float32),
    )(g, ea, w1b, b1, g1, be1)


def _mk_idx(idx_flat, chunks):
    nfull, rem = divmod(chunks, NW)
    cw = nfull + (1 if rem else 0)
    pad = NW * cw - chunks
    ipad = jnp.concatenate([idx_flat, jnp.zeros((pad * CHUNK,), jnp.int32)])
    return ipad.reshape(cw, NW, CHUNK).transpose(1, 0, 2), nfull, rem


def _scatter_body(nfull, rem, n_pad, h_hbm, col4_hbm, zer_hbm, out_hbm,
                  cnt_hbm, ibuf0, ibuf1, buf0, buf1, cnt_v, acc, sem0, sem1,
                  ssem0, ssem1):
    d = h_hbm.shape[1]
    c = lax.axis_index("c")
    s = lax.axis_index("s")
    w = c * NS + s
    n = nfull + jnp.where(w < rem, 1, 0)
    rows_per = n_pad // NS

    pltpu.sync_copy(zer_hbm.at[pl.ds(s * rows_per, rows_per)],
                    acc.at[pl.ds(s * rows_per, rows_per)])

    z16 = jnp.zeros((16,), jnp.float32)

    def zb(i, carry):
        cnt_v[pl.ds(i * 16, 16)] = z16
        return carry

    lax.fori_loop(0, n_pad // 16, zb, 0, unroll=False)
    plsc.subcore_barrier()

    bufs = (buf0, buf1)
    ibufs = (ibuf0, ibuf1)
    sems = (sem0, sem1)
    ssems = (ssem0, ssem1)
    one16 = jnp.ones((16,), jnp.float32)

    def start(i, b):
        @pl.when(i >= 2)
        def _():
            pltpu.make_async_copy(bufs[b], acc.at[ibufs[b].at[0]],
                                  ssems[b]).wait()

        cid = w + NW * i
        pltpu.async_copy(col4_hbm.at[cid], ibufs[b], sems[b])
        pltpu.async_copy(h_hbm.at[pl.ds(cid * CHUNK, CHUNK)], bufs[b], sems[b])

    def finish(i, b):
        cid = w + NW * i
        pltpu.make_async_copy(col4_hbm.at[cid], ibufs[b], sems[b]).wait()
        pltpu.make_async_copy(
            h_hbm.at[pl.ds(cid * CHUNK, CHUNK)], bufs[b], sems[b]
        ).wait()
        pltpu.async_copy(bufs[b], acc.at[ibufs[b].at[0]], ssems[b], add=True)
        for k in range(CHUNK // 16):
            iv = ibufs[b][0, pl.ds(k * 16, 16)]
            plsc.addupdate_scatter(cnt_v, [iv], one16)

    def drain(b):
        ib = n - 1 - ((n - 1 - b) % 2)

        @pl.when(ib >= 0)
        def _():
            pltpu.make_async_copy(bufs[b], acc.at[ibufs[b].at[0]],
                                  ssems[b]).wait()

    cid0 = w
    pltpu.async_copy(col4_hbm.at[cid0], ibufs[0], sems[0])
    pltpu.async_copy(h_hbm.at[pl.ds(cid0 * CHUNK, CHUNK)], bufs[0], sems[0])

    def pair(t, carry):
        i0 = 2 * t
        i1 = i0 + 1

        @pl.when(i1 < n)
        def _():
            start(i1, 1)

        @pl.when(i0 < n)
        def _():
            finish(i0, 0)

        @pl.when(i1 < n)
        def _():
            @pl.when(i1 + 1 < n)
            def _():
                start(i1 + 1, 0)

            finish(i1, 1)

        return carry

    nt = (nfull + 2) // 2
    lax.fori_loop(0, nt, pair, 0, unroll=False)
    drain(0)
    drain(1)

    plsc.subcore_barrier()
    pltpu.sync_copy(acc.at[pl.ds(s * rows_per, rows_per)],
                    out_hbm.at[c, pl.ds(s * rows_per, rows_per)])
    pltpu.sync_copy(cnt_v, cnt_hbm.at[c, s])


def _stage_d(h, col4, n_nodes, nfull, rem):
    d = h.shape[1]
    n_pad = ((n_nodes + 8 * NS - 1) // (8 * NS)) * 8 * NS
    sk = pl.kernel(
        functools.partial(_scatter_body, nfull, rem, n_pad),
        out_type=[
            jax.ShapeDtypeStruct((NC, n_pad, d), jnp.float32),
            jax.ShapeDtypeStruct((NC, NS, n_pad), jnp.float32),
        ],
        mesh=plsc.VectorSubcoreMesh(
            core_axis_name="c", subcore_axis_name="s", num_cores=NC, num_subcores=NS
        ),
        scratch_types=[
            pltpu.VMEM((1, CHUNK), jnp.int32),
            pltpu.VMEM((1, CHUNK), jnp.int32),
            pltpu.VMEM((CHUNK, d), jnp.float32),
            pltpu.VMEM((CHUNK, d), jnp.float32),
            pltpu.VMEM((n_pad,), jnp.float32),
            pltpu.VMEM_SHARED((n_pad, d), jnp.float32),
            pltpu.SemaphoreType.DMA,
            pltpu.SemaphoreType.DMA,
            pltpu.SemaphoreType.DMA,
            pltpu.SemaphoreType.DMA,
        ],
        compiler_params=pltpu.CompilerParams(needs_layout_passes=False),
    )
    zeros = jnp.zeros((n_pad, d), jnp.float32)
    return sk(h, col4, zeros)


def _node_body(nparts, *refs):
    parts_refs = refs[:nparts]
    (cnt_ref, x_ref, bat_ref, u_ref, w2a_ref, w2b_ref, w2c_ref, b2_ref,
     g2_ref, be2_ref, w3_ref, b3_ref, g3_ref, be3_ref, o_ref) = refs[nparts:]
    blk, d = x_ref.shape
    nb = u_ref.shape[1]
    seg = parts_refs[0][0] + parts_refs[0][1]
    for pr in parts_refs[1:]:
        seg = seg + pr[0] + pr[1]
    cnt = jnp.sum(cnt_ref[...], axis=1, keepdims=True)
    agg = seg / jnp.maximum(cnt, 1.0)
    x = x_ref[...]
    oh = (lax.broadcasted_iota(jnp.int32, (blk, nb), 1) == bat_ref[...]).astype(
        jnp.float32
    )
    ub = jnp.sum(oh * u_ref[...], axis=1, keepdims=True)
    t = (
        jnp.dot(x, w2a_ref[...], precision=_HI)
        + jnp.dot(agg, w2b_ref[...], precision=_HI)
        + ub * w2c_ref[...]
        + b2_ref[...]
    )
    t = jnp.maximum(t, 0.0)
    y = _ln(t, g2_ref[...], be2_ref[...])
    y = jnp.dot(y, w3_ref[...], precision=_HI) + b3_ref[...]
    o_ref[...] = _ln(y, g3_ref[...], be3_ref[...]) + x


def _stage_e(parts_list, cnt_t, x_h, bat2, u_row, w2a, w2b, w2c, b2, g2, be2,
             w3, b3, g3, be3, blk):
    n, d = x_h.shape
    nw = cnt_t.shape[1]
    nb = u_row.shape[1]
    nparts = len(parts_list)
    full = lambda i: (0, 0)
    return pl.pallas_call(
        functools.partial(_node_body, nparts),
        grid=(n // blk,),
        in_specs=[
            pl.BlockSpec((NC, blk, d), lambda i: (0, i, 0))
            for _ in range(nparts)
        ] + [
            pl.BlockSpec((blk, nw), lambda i: (i, 0)),
            pl.BlockSpec((blk, d), lambda i: (i, 0)),
            pl.BlockSpec((blk, 1), lambda i: (i, 0)),
            pl.BlockSpec((1, nb), full),
            pl.BlockSpec((d, d), full),
            pl.BlockSpec((d, d), full),
            pl.BlockSpec((1, d), full),
            pl.BlockSpec((1, d), full),
            pl.BlockSpec((1, d), full),
            pl.BlockSpec((1, d), full),
            pl.BlockSpec((d, d), full),
            pl.BlockSpec((1, d), full),
            pl.BlockSpec((1, d), full),
            pl.BlockSpec((1, d), full),
        ],
        out_specs=pl.BlockSpec((blk, d), lambda i: (i, 0)),
        out_shape=jax.ShapeDtypeStruct((n, d), jnp.float32),
    )(*parts_list, cnt_t, x_h, bat2, u_row, w2a, w2b, w2c, b2, g2, be2, w3,
      b3, g3, be3)


def kernel(x_h, edge_index, edge_attr_h, u, batch,
           W1, b1, g1, be1, W2, b2, g2, be2, W3, b3, g3, be3):
    n, d = x_h.shape
    e = edge_index.shape[1]
    nb = u.shape[0]

    row = edge_index[0]
    col = edge_index[1]

    w1a, w1b = W1[:d], W1[d:]
    w2a, w2b, w2c = W2[:d], W2[d:2 * d], W2[2 * d:2 * d + 1]
    r = lambda v: v.reshape(1, d)

    p = _stage_a(x_h, w1a, 2000)

    nsplit = NSPLIT
    eh = e // nsplit
    parts_list, cnts_list = [], []
    for si in range(nsplit):
        a = si * eh
        row3, nfull, rem = _mk_idx(lax.dynamic_slice(row, (a,), (eh,)), eh // CHUNK)
        col4 = lax.dynamic_slice(col, (a,), (eh,)).reshape(eh // CHUNK, 1, CHUNK)
        g = _stage_b(p, row3, eh, nfull, rem)
        h = _stage_c(g, edge_attr_h, w1b.astype(jnp.bfloat16), r(b1), r(g1),
                     r(be1), 2000, off=a)
        parts, cnts = _stage_d(h, col4, n, nfull, rem)
        parts_list.append(parts)
        cnts_list.append(cnts)

    cnt_t = jnp.concatenate([c.reshape(NW, -1) for c in cnts_list], axis=0).T
    out = _stage_e(parts_list, cnt_t, x_h, batch.reshape(n, 1),
                   u.reshape(1, nb), w2a, w2b, w2c, r(b2), r(g2), r(be2), W3,
                   r(b3), r(g3), r(be3), 2000)
    return out
